# baseline (device time: 479978 ns/iter reference)
import jax
import jax.numpy as jnp
from jax import lax
from jax.experimental import pallas as pl
from jax.experimental.pallas import tpu as pltpu

N_DEV = 4
FP8 = jnp.float8_e4m3fn
BM = 128


def _fused_body(x_ref, w_ref, s_ref, o_ref, xg_ref, wg_ref, sems, loc_sems):
    me = lax.axis_index("i")
    left = (me - 1) % N_DEV
    right = (me + 1) % N_DEV

    kx = xg_ref.shape[1] // N_DEV
    kw = wg_ref.shape[0] // N_DEV
    mh = xg_ref.shape[0] // 2
    nh = wg_ref.shape[1] // 2

    step = pl.program_id(0)

    @pl.when(step == 0)
    def _gather():

        cx = pltpu.make_async_copy(
            x_ref, xg_ref.at[:, pl.ds(me * kx, kx)], loc_sems.at[0]
        )
        cw = pltpu.make_async_copy(
            w_ref, wg_ref.at[pl.ds(me * kw, kw), :], loc_sems.at[1]
        )
        cx.start()
        cw.start()
        cx.wait()
        cw.wait()

        all_rdmas = []
        for h in range(N_DEV - 1):
            o_cw = (me - h) % N_DEV
            o_ccw = (me + h) % N_DEV
            rdmas = []
            for s, (src, dst) in enumerate((
                (xg_ref.at[pl.ds(0, mh), pl.ds(o_cw * kx, kx)], right),
                (wg_ref.at[pl.ds(o_cw * kw, kw), pl.ds(0, nh)], right),
                (xg_ref.at[pl.ds(mh, mh), pl.ds(o_ccw * kx, kx)], left),
                (wg_ref.at[pl.ds(o_ccw * kw, kw), pl.ds(nh, nh)], left),
            )):
                r = pltpu.make_async_remote_copy(
                    src_ref=src,
                    dst_ref=src,
                    send_sem=sems.at[2 * s, h],
                    recv_sem=sems.at[2 * s + 1, h],
                    device_id=(dst,),
                    device_id_type=pl.DeviceIdType.MESH,
                )
                r.start()
                rdmas.append(r)
            for r in rdmas:
                r.wait_recv()
            all_rdmas.extend(rdmas)
        for r in all_rdmas:
            r.wait_send()

    acc = jnp.dot(
        xg_ref[pl.ds(step * BM, BM), :], wg_ref[...],
        preferred_element_type=jnp.float32,
    )
    o_ref[...] = jnp.maximum(acc * s_ref[0, 0], 0.0)


def kernel(x, w_mat, scale_x, scale_w):
    if x.dtype != FP8:
        x = x.astype(FP8)
    if w_mat.dtype != FP8:
        w_mat = w_mat.astype(FP8)
    s = (scale_x.astype(jnp.float32) * scale_w.astype(jnp.float32)).reshape(1, 1)

    m, kx = x.shape
    kw, n = w_mat.shape
    k = kx * N_DEV

    y = pl.pallas_call(
        _fused_body,
        grid=(m // BM,),
        in_specs=[
            pl.BlockSpec(memory_space=pl.ANY),
            pl.BlockSpec(memory_space=pl.ANY),
            pl.BlockSpec(memory_space=pltpu.SMEM),
        ],
        out_specs=pl.BlockSpec((BM, n), lambda i: (i, 0)),
        out_shape=jax.ShapeDtypeStruct((m, n), jnp.float32),
        scratch_shapes=[
            pltpu.VMEM((m, k), FP8),
            pltpu.VMEM((k, n), FP8),
            pltpu.SemaphoreType.DMA((8, N_DEV - 1)),
            pltpu.SemaphoreType.DMA((2,)),
        ],
        compiler_params=pltpu.CompilerParams(
            vmem_limit_bytes=100 * 1024 * 1024
        ),
    )(x, w_mat, s)
    return y


# device time: 411872 ns/iter; 1.1654x vs baseline; 1.1654x over previous
import jax
import jax.numpy as jnp
from jax import lax
from jax.experimental import pallas as pl
from jax.experimental.pallas import tpu as pltpu

N_DEV = 4
FP8 = jnp.float8_e4m3fn


def _ag_body(x_ref, w_ref, xg_ref, wg_ref, sems):
    me = lax.axis_index("i")
    left = (me - 1) % N_DEV
    right = (me + 1) % N_DEV

    barrier = pltpu.get_barrier_semaphore()
    for nbr in (left, right):
        pl.semaphore_signal(
            barrier, inc=1, device_id=(nbr,),
            device_id_type=pl.DeviceIdType.MESH,
        )
    pl.semaphore_wait(barrier, 2)

    kx = x_ref.shape[1]
    kw = w_ref.shape[0]
    mh = x_ref.shape[0] // 2
    nh = w_ref.shape[1] // 2

    xg_ref[:, pl.ds(me * kx, kx)] = x_ref[...]
    wg_ref[pl.ds(me * kw, kw), :] = w_ref[...]

    all_rdmas = []
    for h in range(N_DEV - 1):
        o_cw = (me - h) % N_DEV
        o_ccw = (me + h) % N_DEV
        rdmas = []
        for s, (src, dst) in enumerate((
            (xg_ref.at[pl.ds(0, mh), pl.ds(o_cw * kx, kx)], right),
            (wg_ref.at[pl.ds(o_cw * kw, kw), pl.ds(0, nh)], right),
            (xg_ref.at[pl.ds(mh, mh), pl.ds(o_ccw * kx, kx)], left),
            (wg_ref.at[pl.ds(o_ccw * kw, kw), pl.ds(nh, nh)], left),
        )):
            r = pltpu.make_async_remote_copy(
                src_ref=src,
                dst_ref=src,
                send_sem=sems.at[2 * s, h],
                recv_sem=sems.at[2 * s + 1, h],
                device_id=(dst,),
                device_id_type=pl.DeviceIdType.MESH,
            )
            r.start()
            rdmas.append(r)
        for r in rdmas:
            r.wait_recv()
        all_rdmas.extend(rdmas)
    for r in all_rdmas:
        r.wait_send()


def _gemm_body(xg_ref, wg_ref, s_ref, o_ref):
    acc = jnp.dot(xg_ref[...], wg_ref[...], preferred_element_type=jnp.float32)
    o_ref[...] = jnp.maximum(acc * s_ref[0, 0], 0.0)


def kernel(x, w_mat, scale_x, scale_w):
    if x.dtype != FP8:
        x = x.astype(FP8)
    if w_mat.dtype != FP8:
        w_mat = w_mat.astype(FP8)
    s = (scale_x.astype(jnp.float32) * scale_w.astype(jnp.float32)).reshape(1, 1)

    m, kx = x.shape
    kw, n = w_mat.shape
    k = kx * N_DEV

    xg, wg = pl.pallas_call(
        _ag_body,
        out_shape=[
            jax.ShapeDtypeStruct((m, k), FP8),
            jax.ShapeDtypeStruct((k, n), FP8),
        ],
        in_specs=[
            pl.BlockSpec(memory_space=pltpu.VMEM),
            pl.BlockSpec(memory_space=pltpu.VMEM),
        ],
        out_specs=[
            pl.BlockSpec(memory_space=pltpu.VMEM),
            pl.BlockSpec(memory_space=pltpu.VMEM),
        ],
        scratch_shapes=[
            pltpu.SemaphoreType.DMA((8, N_DEV - 1)),
        ],
        compiler_params=pltpu.CompilerParams(
            collective_id=0, vmem_limit_bytes=100 * 1024 * 1024
        ),
    )(x, w_mat)

    bm, bn = 512, 4096
    y = pl.pallas_call(
        _gemm_body,
        grid=(n // bn, m // bm),
        in_specs=[
            pl.BlockSpec((bm, k), lambda i, j: (j, 0)),
            pl.BlockSpec((k, bn), lambda i, j: (0, i)),
            pl.BlockSpec(memory_space=pltpu.SMEM),
        ],
        out_specs=pl.BlockSpec((bm, bn), lambda i, j: (j, i)),
        out_shape=jax.ShapeDtypeStruct((m, n), jnp.float32),
        compiler_params=pltpu.CompilerParams(
            vmem_limit_bytes=100 * 1024 * 1024
        ),
    )(xg, wg, s)
    return y
